# baseline (speedup 1.0000x reference)
import functools
import jax
import jax.numpy as jnp
from jax import lax
from jax.experimental import pallas as pl
from jax.experimental.pallas import tpu as pltpu
from jax.experimental.pallas import tpu_sc as plsc

_NC = 2
_NS = 16
_NW = _NC * _NS
_L = 16


def _sc_body(nb, q_hbm, k_hbm, v_hbm, eijt_hbm, mlp_hbm, idx_hbm, out_hbm,
             eij_v, mlp_v, idx_v,
             qb0, qb1, qb2, kb0, kb1, vb0, vb1, sca_v,
             sq0, sq1, sq2, sk0, sk1, sv0, sv1, so0, so1, so2):
    B, N, C = q_hbm.shape
    E = eijt_hbm.shape[1]
    nch = C // _L
    bpw = nb // _NW
    wid = lax.axis_index("s") * _NC + lax.axis_index("c")
    base = wid * bpw

    qbufs = [qb0, qb1, qb2]
    kbufs = [kb0, kb1]
    vbufs = [vb0, vb1]
    sq = [sq0, sq1, sq2]
    sk = [sk0, sk1]
    sv = [sv0, sv1]
    so = [so0, so1, so2]

    pltpu.sync_copy(eijt_hbm, eij_v)
    pltpu.sync_copy(mlp_hbm, mlp_v)
    pltpu.sync_copy(idx_hbm, idx_v)

    row0 = jnp.zeros((_L,), jnp.int32)
    row1 = jnp.full((_L,), 1, jnp.int32)
    row2 = jnp.full((_L,), 2, jnp.int32)

    w1s = mlp_v[0, pl.ds(0, _L)]
    w1d = mlp_v[1, pl.ds(0, _L)]
    b1r = mlp_v[2, pl.ds(0, _L)]
    w2r = mlp_v[3, pl.ds(0, _L)]
    b2r = mlp_v[4, pl.ds(0, _L)]
    for t in range(E // _L):
        s_vec = eij_v[0, pl.ds(t * _L, _L)]
        d_vec = eij_v[1, pl.ds(t * _L, _L)]
        acc = jnp.zeros((_L,), jnp.float32) + b2r[0]
        for j in range(16):
            h = jnp.maximum(s_vec * w1s[j] + d_vec * w1d[j] + b1r[j], 0.0)
            acc = acc + h * w2r[j]
        sca_v[0, pl.ds(t * _L, _L)] = acc

    for t in range(N // _L):
        sl = pl.ds(t * _L, _L)
        sca_v[1, sl] = plsc.load_gather(sca_v, [row0, idx_v[2, sl]])
        sca_v[2, sl] = plsc.load_gather(sca_v, [row0, idx_v[3, sl]])

    def node_compute(n, qb, kb, vb):
        nsplat = jnp.full((_L,), n, jnp.int32)
        p0 = plsc.load_gather(idx_v, [row0, nsplat])[0]
        p1 = plsc.load_gather(idx_v, [row1, nsplat])[0]
        a0a = jnp.zeros((_L,), jnp.float32)
        a0b = jnp.zeros((_L,), jnp.float32)
        a1a = jnp.zeros((_L,), jnp.float32)
        a1b = jnp.zeros((_L,), jnp.float32)
        for ct in range(0, nch, 2):
            qc = qb[n, pl.ds(ct * _L, _L)]
            a0a = a0a + qc * kb[p0, pl.ds(ct * _L, _L)]
            a1a = a1a + qc * kb[p1, pl.ds(ct * _L, _L)]
            qd = qb[n, pl.ds((ct + 1) * _L, _L)]
            a0b = a0b + qd * kb[p0, pl.ds((ct + 1) * _L, _L)]
            a1b = a1b + qd * kb[p1, pl.ds((ct + 1) * _L, _L)]
        s0 = jnp.sum(a0a + a0b, axis=0)
        s1 = jnp.sum(a1a + a1b, axis=0)
        l0 = plsc.load_gather(sca_v, [row1, nsplat]) + s0
        l1 = plsc.load_gather(sca_v, [row2, nsplat]) + s1
        m = jnp.maximum(l0, l1)
        e0 = jnp.exp(l0 - m)
        e1 = jnp.exp(l1 - m)
        inv = 1.0 / (e0 + e1)
        a0 = e0 * inv
        a1 = e1 * inv
        for ct in range(nch):
            sl = pl.ds(ct * _L, _L)
            qb[n, sl] = vb[p0, sl] * a0 + vb[p1, sl] * a1

    def issue_in(bi):
        b = base + bi
        return [
            pltpu.async_copy(q_hbm.at[b], qbufs[bi % 3], sq[bi % 3]),
            pltpu.async_copy(k_hbm.at[b], kbufs[bi % 2], sk[bi % 2]),
            pltpu.async_copy(v_hbm.at[b], vbufs[bi % 2], sv[bi % 2]),
        ]

    inh = {0: issue_in(0)}
    if bpw > 1:
        inh[1] = issue_in(1)
    outh = {}
    out_waited = set()
    for bi in range(bpw):
        for h in inh.pop(bi):
            h.wait()
        qb = qbufs[bi % 3]
        kb = kbufs[bi % 2]
        vb = vbufs[bi % 2]

        @plsc.parallel_loop(0, N, 1, unroll=1)
        def _nloop(n, qb=qb, kb=kb, vb=vb):
            node_compute(n, qb, kb, vb)
        outh[bi] = pltpu.async_copy(qb, out_hbm.at[base + bi], so[bi % 3])
        if bi + 2 < bpw:
            if bi - 1 >= 0 and (bi - 1) not in out_waited:
                outh[bi - 1].wait()
                out_waited.add(bi - 1)
            inh[bi + 2] = issue_in(bi + 2)
    for bi in range(bpw):
        if bi not in out_waited:
            outh[bi].wait()


def _sc_call(nb, Q, K, V, eijT, mlp, idx):
    B, N, C = Q.shape
    E = eijT.shape[1]
    mesh = plsc.VectorSubcoreMesh(
        core_axis_name="c", subcore_axis_name="s", num_cores=_NC, num_subcores=_NS
    )
    buf = pltpu.VMEM((N, C), jnp.float32)
    return pl.kernel(
        functools.partial(_sc_body, nb),
        out_type=jax.ShapeDtypeStruct((nb, N, C), jnp.float32),
        mesh=mesh,
        compiler_params=pltpu.CompilerParams(needs_layout_passes=False),
        scratch_types=[
            pltpu.VMEM((2, E), jnp.float32),
            pltpu.VMEM((8, 16), jnp.float32),
            pltpu.VMEM((4, N), jnp.int32),
            buf, buf, buf, buf, buf, buf, buf,
            pltpu.VMEM((8, E), jnp.float32),
        ] + [pltpu.SemaphoreType.DMA] * 10,
    )(Q, K, V, eijT, mlp, idx)


_SC_BATCHES = 128


def _tc_body(qref, kref, vref, eijt, w1, b1c, w2, b2c, p0, p1, g0, g1, oref):
    h = jnp.maximum(
        jnp.dot(w1[...], eijt[...], preferred_element_type=jnp.float32) + b1c[...],
        0.0,
    )
    bias = jnp.dot(w2[...], h, preferred_element_type=jnp.float32) + b2c[...]
    bias0 = jnp.sum(g0[...] * bias, axis=1)
    bias1 = jnp.sum(g1[...] * bias, axis=1)

    q = qref[...]
    k = kref[...]
    v = vref[...]
    dn = (((2,), (2,)), ((0,), (0,)))
    s = jax.lax.dot_general(
        q, k, dn,
        precision=jax.lax.Precision.HIGHEST,
        preferred_element_type=jnp.float32,
    )
    l0 = jnp.sum(s * p0[...][None], axis=2) + bias0[None]
    l1 = jnp.sum(s * p1[...][None], axis=2) + bias1[None]
    m = jnp.maximum(l0, l1)
    e0 = jnp.exp(l0 - m)
    e1 = jnp.exp(l1 - m)
    inv = 1.0 / (e0 + e1)
    wmix = (e0 * inv)[:, :, None] * p0[...][None] + (e1 * inv)[:, :, None] * p1[...][None]
    dn2 = (((2,), (1,)), ((0,), (0,)))
    oref[...] = jax.lax.dot_general(wmix, v, dn2, preferred_element_type=jnp.float32)


def _tc_call(skip, Q, K, V, eijT, W1, b1c, W2, b2c, P0, P1, G0, G1):
    B, N, C = Q.shape
    E = eijT.shape[1]
    H = W1.shape[0]
    bs = 16
    off = skip // bs
    grid = ((B - skip) // bs,)
    full = lambda i: (0, 0)
    blk = pl.BlockSpec((bs, N, C), lambda i: (i + off, 0, 0))
    return pl.pallas_call(
        _tc_body,
        grid=grid,
        in_specs=[
            blk,
            blk,
            blk,
            pl.BlockSpec((2, E), full),
            pl.BlockSpec((H, 2), full),
            pl.BlockSpec((H, 1), full),
            pl.BlockSpec((1, H), full),
            pl.BlockSpec((1, 1), full),
            pl.BlockSpec((N, N), full),
            pl.BlockSpec((N, N), full),
            pl.BlockSpec((N, E), full),
            pl.BlockSpec((N, E), full),
        ],
        out_specs=blk,
        out_shape=jax.ShapeDtypeStruct((B, N, C), jnp.float32),
    )(Q, K, V, eijT, W1, b1c, W2, b2c, P0, P1, G0, G1)


def kernel(Q, K, V, eij, W1, b1, W2, b2, src, dst):
    B, N, C = Q.shape
    E = src.shape[0]
    H = W1.shape[0]

    order = jnp.argsort(dst.astype(jnp.int32))
    psrc = src.astype(jnp.int32)[order]
    p0i = psrc[0::2]
    p1i = psrc[1::2]
    e0i = order[0::2].astype(jnp.int32)
    e1i = order[1::2].astype(jnp.int32)
    idx = jnp.stack([p0i, p1i, e0i, e1i]).astype(jnp.int32)

    eijT = eij.T
    mlp = (
        jnp.zeros((8, 16), jnp.float32)
        .at[0].set(W1[:, 0])
        .at[1].set(W1[:, 1])
        .at[2].set(b1)
        .at[3].set(W2[0])
        .at[4, 0].set(b2[0])
    )

    bsc = _SC_BATCHES
    if bsc >= B:
        return _sc_call(B, Q, K, V, eijT, mlp, idx)

    ar = jnp.arange(N, dtype=jnp.int32)
    are = jnp.arange(E, dtype=jnp.int32)
    P0 = (p0i[:, None] == ar[None, :]).astype(jnp.float32)
    P1 = (p1i[:, None] == ar[None, :]).astype(jnp.float32)
    G0 = (e0i[:, None] == are[None, :]).astype(jnp.float32)
    G1 = (e1i[:, None] == are[None, :]).astype(jnp.float32)
    b1c = b1.reshape(H, 1)
    b2c = b2.reshape(1, 1)

    out_tc = _tc_call(bsc, Q, K, V, eijT, W1, b1c, W2, b2c, P0, P1, G0, G1)
    out_sc = _sc_call(bsc, Q, K, V, eijT, mlp, idx)
    return lax.dynamic_update_slice(out_tc, out_sc, (0, 0, 0))


# speedup vs baseline: 1.1424x; 1.1424x over previous
import functools
import jax
import jax.numpy as jnp
from jax import lax
from jax.experimental import pallas as pl
from jax.experimental.pallas import tpu as pltpu
from jax.experimental.pallas import tpu_sc as plsc

_NC = 2
_NS = 16
_NW = _NC * _NS
_L = 16


def _sc_body(nb, q_hbm, k_hbm, v_hbm, eijt_hbm, mlp_hbm, idx_hbm, out_hbm,
             eij_v, mlp_v, idx_v,
             qb0, qb1, qb2, kb0, kb1, vb0, vb1, sca_v,
             sq0, sq1, sq2, sk0, sk1, sv0, sv1, so0, so1, so2):
    B, N, C = q_hbm.shape
    E = eijt_hbm.shape[1]
    nch = C // _L
    bpw = nb // _NW
    wid = lax.axis_index("s") * _NC + lax.axis_index("c")
    base = wid * bpw

    qbufs = [qb0, qb1, qb2]
    kbufs = [kb0, kb1]
    vbufs = [vb0, vb1]
    sq = [sq0, sq1, sq2]
    sk = [sk0, sk1]
    sv = [sv0, sv1]
    so = [so0, so1, so2]

    pltpu.sync_copy(eijt_hbm, eij_v)
    pltpu.sync_copy(mlp_hbm, mlp_v)
    pltpu.sync_copy(idx_hbm, idx_v)

    row0 = jnp.zeros((_L,), jnp.int32)
    row1 = jnp.full((_L,), 1, jnp.int32)
    row2 = jnp.full((_L,), 2, jnp.int32)

    w1s = mlp_v[0, pl.ds(0, _L)]
    w1d = mlp_v[1, pl.ds(0, _L)]
    b1r = mlp_v[2, pl.ds(0, _L)]
    w2r = mlp_v[3, pl.ds(0, _L)]
    b2r = mlp_v[4, pl.ds(0, _L)]
    for t in range(E // _L):
        s_vec = eij_v[0, pl.ds(t * _L, _L)]
        d_vec = eij_v[1, pl.ds(t * _L, _L)]
        acc = jnp.zeros((_L,), jnp.float32) + b2r[0]
        for j in range(16):
            h = jnp.maximum(s_vec * w1s[j] + d_vec * w1d[j] + b1r[j], 0.0)
            acc = acc + h * w2r[j]
        sca_v[0, pl.ds(t * _L, _L)] = acc

    for t in range(N // _L):
        sl = pl.ds(t * _L, _L)
        sca_v[1, sl] = plsc.load_gather(sca_v, [row0, idx_v[2, sl]])
        sca_v[2, sl] = plsc.load_gather(sca_v, [row0, idx_v[3, sl]])

    def node_compute(n, qb, kb, vb):
        nsplat = jnp.full((_L,), n, jnp.int32)
        p0 = plsc.load_gather(idx_v, [row0, nsplat])[0]
        p1 = plsc.load_gather(idx_v, [row1, nsplat])[0]
        a0a = jnp.zeros((_L,), jnp.float32)
        a0b = jnp.zeros((_L,), jnp.float32)
        a1a = jnp.zeros((_L,), jnp.float32)
        a1b = jnp.zeros((_L,), jnp.float32)
        for ct in range(0, nch, 2):
            qc = qb[n, pl.ds(ct * _L, _L)]
            a0a = a0a + qc * kb[p0, pl.ds(ct * _L, _L)]
            a1a = a1a + qc * kb[p1, pl.ds(ct * _L, _L)]
            qd = qb[n, pl.ds((ct + 1) * _L, _L)]
            a0b = a0b + qd * kb[p0, pl.ds((ct + 1) * _L, _L)]
            a1b = a1b + qd * kb[p1, pl.ds((ct + 1) * _L, _L)]
        s0 = jnp.sum(a0a + a0b, axis=0)
        s1 = jnp.sum(a1a + a1b, axis=0)
        l0 = plsc.load_gather(sca_v, [row1, nsplat]) + s0
        l1 = plsc.load_gather(sca_v, [row2, nsplat]) + s1
        m = jnp.maximum(l0, l1)
        e0 = jnp.exp(l0 - m)
        e1 = jnp.exp(l1 - m)
        inv = 1.0 / (e0 + e1)
        a0 = e0 * inv
        a1 = e1 * inv
        for ct in range(nch):
            sl = pl.ds(ct * _L, _L)
            qb[n, sl] = vb[p0, sl] * a0 + vb[p1, sl] * a1

    def issue_in(bi):
        b = base + bi
        return [
            pltpu.async_copy(q_hbm.at[b], qbufs[bi % 3], sq[bi % 3]),
            pltpu.async_copy(k_hbm.at[b], kbufs[bi % 2], sk[bi % 2]),
            pltpu.async_copy(v_hbm.at[b], vbufs[bi % 2], sv[bi % 2]),
        ]

    inh = {0: issue_in(0)}
    if bpw > 1:
        inh[1] = issue_in(1)
    outh = {}
    out_waited = set()
    for bi in range(bpw):
        for h in inh.pop(bi):
            h.wait()
        qb = qbufs[bi % 3]
        kb = kbufs[bi % 2]
        vb = vbufs[bi % 2]

        @plsc.parallel_loop(0, N, 1, unroll=1)
        def _nloop(n, qb=qb, kb=kb, vb=vb):
            node_compute(n, qb, kb, vb)
        outh[bi] = pltpu.async_copy(qb, out_hbm.at[base + bi], so[bi % 3])
        if bi + 2 < bpw:
            if bi - 1 >= 0 and (bi - 1) not in out_waited:
                outh[bi - 1].wait()
                out_waited.add(bi - 1)
            inh[bi + 2] = issue_in(bi + 2)
    for bi in range(bpw):
        if bi not in out_waited:
            outh[bi].wait()


def _sc_call(nb, Q, K, V, eijT, mlp, idx):
    B, N, C = Q.shape
    E = eijT.shape[1]
    mesh = plsc.VectorSubcoreMesh(
        core_axis_name="c", subcore_axis_name="s", num_cores=_NC, num_subcores=_NS
    )
    buf = pltpu.VMEM((N, C), jnp.float32)
    return pl.kernel(
        functools.partial(_sc_body, nb),
        out_type=jax.ShapeDtypeStruct((nb, N, C), jnp.float32),
        mesh=mesh,
        compiler_params=pltpu.CompilerParams(needs_layout_passes=False),
        scratch_types=[
            pltpu.VMEM((2, E), jnp.float32),
            pltpu.VMEM((8, 16), jnp.float32),
            pltpu.VMEM((4, N), jnp.int32),
            buf, buf, buf, buf, buf, buf, buf,
            pltpu.VMEM((8, E), jnp.float32),
        ] + [pltpu.SemaphoreType.DMA] * 10,
    )(Q, K, V, eijT, mlp, idx)


_SC_BATCHES = 64


def _tc_body(qref, kref, vref, eijt, w1, b1c, w2, b2c, p0, p1, g0, g1, oref):
    h = jnp.maximum(
        jnp.dot(w1[...], eijt[...], preferred_element_type=jnp.float32) + b1c[...],
        0.0,
    )
    bias = jnp.dot(w2[...], h, preferred_element_type=jnp.float32) + b2c[...]
    bias0 = jnp.sum(g0[...] * bias, axis=1)
    bias1 = jnp.sum(g1[...] * bias, axis=1)

    q = qref[...]
    k = kref[...]
    v = vref[...]
    dn = (((2,), (2,)), ((0,), (0,)))
    s = jax.lax.dot_general(
        q, k, dn,
        precision=jax.lax.Precision.HIGHEST,
        preferred_element_type=jnp.float32,
    )
    l0 = jnp.sum(s * p0[...][None], axis=2) + bias0[None]
    l1 = jnp.sum(s * p1[...][None], axis=2) + bias1[None]
    m = jnp.maximum(l0, l1)
    e0 = jnp.exp(l0 - m)
    e1 = jnp.exp(l1 - m)
    inv = 1.0 / (e0 + e1)
    wmix = (e0 * inv)[:, :, None] * p0[...][None] + (e1 * inv)[:, :, None] * p1[...][None]
    dn2 = (((2,), (1,)), ((0,), (0,)))
    oref[...] = jax.lax.dot_general(wmix, v, dn2, preferred_element_type=jnp.float32)


def _tc_call(skip, Q, K, V, eijT, W1, b1c, W2, b2c, P0, P1, G0, G1):
    B, N, C = Q.shape
    E = eijT.shape[1]
    H = W1.shape[0]
    bs = 16
    off = skip // bs
    grid = ((B - skip) // bs,)
    full = lambda i: (0, 0)
    blk = pl.BlockSpec((bs, N, C), lambda i: (i + off, 0, 0))
    return pl.pallas_call(
        _tc_body,
        grid=grid,
        in_specs=[
            blk,
            blk,
            blk,
            pl.BlockSpec((2, E), full),
            pl.BlockSpec((H, 2), full),
            pl.BlockSpec((H, 1), full),
            pl.BlockSpec((1, H), full),
            pl.BlockSpec((1, 1), full),
            pl.BlockSpec((N, N), full),
            pl.BlockSpec((N, N), full),
            pl.BlockSpec((N, E), full),
            pl.BlockSpec((N, E), full),
        ],
        out_specs=blk,
        out_shape=jax.ShapeDtypeStruct((B, N, C), jnp.float32),
    )(Q, K, V, eijT, W1, b1c, W2, b2c, P0, P1, G0, G1)


def kernel(Q, K, V, eij, W1, b1, W2, b2, src, dst):
    B, N, C = Q.shape
    E = src.shape[0]
    H = W1.shape[0]

    order = jnp.argsort(dst.astype(jnp.int32))
    psrc = src.astype(jnp.int32)[order]
    p0i = psrc[0::2]
    p1i = psrc[1::2]
    e0i = order[0::2].astype(jnp.int32)
    e1i = order[1::2].astype(jnp.int32)
    idx = jnp.stack([p0i, p1i, e0i, e1i]).astype(jnp.int32)

    eijT = eij.T
    mlp = (
        jnp.zeros((8, 16), jnp.float32)
        .at[0].set(W1[:, 0])
        .at[1].set(W1[:, 1])
        .at[2].set(b1)
        .at[3].set(W2[0])
        .at[4, 0].set(b2[0])
    )

    bsc = _SC_BATCHES
    if bsc >= B:
        return _sc_call(B, Q, K, V, eijT, mlp, idx)

    ar = jnp.arange(N, dtype=jnp.int32)
    are = jnp.arange(E, dtype=jnp.int32)
    P0 = (p0i[:, None] == ar[None, :]).astype(jnp.float32)
    P1 = (p1i[:, None] == ar[None, :]).astype(jnp.float32)
    G0 = (e0i[:, None] == are[None, :]).astype(jnp.float32)
    G1 = (e1i[:, None] == are[None, :]).astype(jnp.float32)
    b1c = b1.reshape(H, 1)
    b2c = b2.reshape(1, 1)

    out_tc = _tc_call(bsc, Q, K, V, eijT, W1, b1c, W2, b2c, P0, P1, G0, G1)
    out_sc = _sc_call(bsc, Q, K, V, eijT, mlp, idx)
    return lax.dynamic_update_slice(out_tc, out_sc, (0, 0, 0))


# speedup vs baseline: 1.1431x; 1.0007x over previous
import functools
import jax
import jax.numpy as jnp
from jax import lax
from jax.experimental import pallas as pl
from jax.experimental.pallas import tpu as pltpu
from jax.experimental.pallas import tpu_sc as plsc

_NC = 2
_NS = 16
_NW = _NC * _NS
_L = 16


def _sc_body(nb, q_hbm, k_hbm, v_hbm, eijt_hbm, mlp_hbm, idx_hbm, out_hbm,
             eij_v, mlp_v, idx_v,
             qb0, qb1, qb2, kb0, kb1, vb0, vb1, sca_v,
             sq0, sq1, sq2, sk0, sk1, sv0, sv1, so0, so1, so2):
    B, N, C = q_hbm.shape
    E = eijt_hbm.shape[1]
    nch = C // _L
    bpw = nb // _NW
    wid = lax.axis_index("s") * _NC + lax.axis_index("c")
    base = wid * bpw

    qbufs = [qb0, qb1, qb2]
    kbufs = [kb0, kb1]
    vbufs = [vb0, vb1]
    sq = [sq0, sq1, sq2]
    sk = [sk0, sk1]
    sv = [sv0, sv1]
    so = [so0, so1, so2]

    pltpu.sync_copy(eijt_hbm, eij_v)
    pltpu.sync_copy(mlp_hbm, mlp_v)
    pltpu.sync_copy(idx_hbm, idx_v)

    row0 = jnp.zeros((_L,), jnp.int32)
    row1 = jnp.full((_L,), 1, jnp.int32)
    row2 = jnp.full((_L,), 2, jnp.int32)

    w1s = mlp_v[0, pl.ds(0, _L)]
    w1d = mlp_v[1, pl.ds(0, _L)]
    b1r = mlp_v[2, pl.ds(0, _L)]
    w2r = mlp_v[3, pl.ds(0, _L)]
    b2r = mlp_v[4, pl.ds(0, _L)]
    for t in range(E // _L):
        s_vec = eij_v[0, pl.ds(t * _L, _L)]
        d_vec = eij_v[1, pl.ds(t * _L, _L)]
        acc = jnp.zeros((_L,), jnp.float32) + b2r[0]
        for j in range(16):
            h = jnp.maximum(s_vec * w1s[j] + d_vec * w1d[j] + b1r[j], 0.0)
            acc = acc + h * w2r[j]
        sca_v[0, pl.ds(t * _L, _L)] = acc

    for t in range(N // _L):
        sl = pl.ds(t * _L, _L)
        sca_v[1, sl] = plsc.load_gather(sca_v, [row0, idx_v[2, sl]])
        sca_v[2, sl] = plsc.load_gather(sca_v, [row0, idx_v[3, sl]])

    def node_compute(n, qb, kb, vb):
        nsplat = jnp.full((_L,), n, jnp.int32)
        p0 = plsc.load_gather(idx_v, [row0, nsplat])[0]
        p1 = plsc.load_gather(idx_v, [row1, nsplat])[0]
        acc0 = [jnp.zeros((_L,), jnp.float32) for _ in range(4)]
        acc1 = [jnp.zeros((_L,), jnp.float32) for _ in range(4)]
        for ct in range(0, nch, 4):
            for u in range(4):
                sl = pl.ds((ct + u) * _L, _L)
                qc = qb[n, sl]
                acc0[u] = acc0[u] + qc * kb[p0, sl]
                acc1[u] = acc1[u] + qc * kb[p1, sl]
        s0 = jnp.sum((acc0[0] + acc0[1]) + (acc0[2] + acc0[3]), axis=0)
        s1 = jnp.sum((acc1[0] + acc1[1]) + (acc1[2] + acc1[3]), axis=0)
        l0 = plsc.load_gather(sca_v, [row1, nsplat]) + s0
        l1 = plsc.load_gather(sca_v, [row2, nsplat]) + s1
        m = jnp.maximum(l0, l1)
        e0 = jnp.exp(l0 - m)
        e1 = jnp.exp(l1 - m)
        inv = 1.0 / (e0 + e1)
        a0 = e0 * inv
        a1 = e1 * inv
        for ct in range(nch):
            sl = pl.ds(ct * _L, _L)
            qb[n, sl] = vb[p0, sl] * a0 + vb[p1, sl] * a1

    def issue_in(bi):
        b = base + bi
        return [
            pltpu.async_copy(q_hbm.at[b], qbufs[bi % 3], sq[bi % 3]),
            pltpu.async_copy(k_hbm.at[b], kbufs[bi % 2], sk[bi % 2]),
            pltpu.async_copy(v_hbm.at[b], vbufs[bi % 2], sv[bi % 2]),
        ]

    inh = {0: issue_in(0)}
    if bpw > 1:
        inh[1] = issue_in(1)
    outh = {}
    out_waited = set()
    for bi in range(bpw):
        for h in inh.pop(bi):
            h.wait()
        qb = qbufs[bi % 3]
        kb = kbufs[bi % 2]
        vb = vbufs[bi % 2]

        @plsc.parallel_loop(0, N, 1, unroll=1)
        def _nloop(n, qb=qb, kb=kb, vb=vb):
            node_compute(n, qb, kb, vb)
        outh[bi] = pltpu.async_copy(qb, out_hbm.at[base + bi], so[bi % 3])
        if bi + 2 < bpw:
            if bi - 1 >= 0 and (bi - 1) not in out_waited:
                outh[bi - 1].wait()
                out_waited.add(bi - 1)
            inh[bi + 2] = issue_in(bi + 2)
    for bi in range(bpw):
        if bi not in out_waited:
            outh[bi].wait()


def _sc_call(nb, Q, K, V, eijT, mlp, idx):
    B, N, C = Q.shape
    E = eijT.shape[1]
    mesh = plsc.VectorSubcoreMesh(
        core_axis_name="c", subcore_axis_name="s", num_cores=_NC, num_subcores=_NS
    )
    buf = pltpu.VMEM((N, C), jnp.float32)
    return pl.kernel(
        functools.partial(_sc_body, nb),
        out_type=jax.ShapeDtypeStruct((nb, N, C), jnp.float32),
        mesh=mesh,
        compiler_params=pltpu.CompilerParams(needs_layout_passes=False),
        scratch_types=[
            pltpu.VMEM((2, E), jnp.float32),
            pltpu.VMEM((8, 16), jnp.float32),
            pltpu.VMEM((4, N), jnp.int32),
            buf, buf, buf, buf, buf, buf, buf,
            pltpu.VMEM((8, E), jnp.float32),
        ] + [pltpu.SemaphoreType.DMA] * 10,
    )(Q, K, V, eijT, mlp, idx)


_SC_BATCHES = 64


def _tc_body(qref, kref, vref, eijt, w1, b1c, w2, b2c, p0, p1, g0, g1, oref):
    h = jnp.maximum(
        jnp.dot(w1[...], eijt[...], preferred_element_type=jnp.float32) + b1c[...],
        0.0,
    )
    bias = jnp.dot(w2[...], h, preferred_element_type=jnp.float32) + b2c[...]
    bias0 = jnp.sum(g0[...] * bias, axis=1)
    bias1 = jnp.sum(g1[...] * bias, axis=1)

    q = qref[...]
    k = kref[...]
    v = vref[...]
    dn = (((2,), (2,)), ((0,), (0,)))
    s = jax.lax.dot_general(
        q, k, dn,
        precision=jax.lax.Precision.HIGHEST,
        preferred_element_type=jnp.float32,
    )
    l0 = jnp.sum(s * p0[...][None], axis=2) + bias0[None]
    l1 = jnp.sum(s * p1[...][None], axis=2) + bias1[None]
    m = jnp.maximum(l0, l1)
    e0 = jnp.exp(l0 - m)
    e1 = jnp.exp(l1 - m)
    inv = 1.0 / (e0 + e1)
    wmix = (e0 * inv)[:, :, None] * p0[...][None] + (e1 * inv)[:, :, None] * p1[...][None]
    dn2 = (((2,), (1,)), ((0,), (0,)))
    oref[...] = jax.lax.dot_general(wmix, v, dn2, preferred_element_type=jnp.float32)


def _tc_call(skip, Q, K, V, eijT, W1, b1c, W2, b2c, P0, P1, G0, G1):
    B, N, C = Q.shape
    E = eijT.shape[1]
    H = W1.shape[0]
    bs = 16
    off = skip // bs
    grid = ((B - skip) // bs,)
    full = lambda i: (0, 0)
    blk = pl.BlockSpec((bs, N, C), lambda i: (i + off, 0, 0))
    return pl.pallas_call(
        _tc_body,
        grid=grid,
        in_specs=[
            blk,
            blk,
            blk,
            pl.BlockSpec((2, E), full),
            pl.BlockSpec((H, 2), full),
            pl.BlockSpec((H, 1), full),
            pl.BlockSpec((1, H), full),
            pl.BlockSpec((1, 1), full),
            pl.BlockSpec((N, N), full),
            pl.BlockSpec((N, N), full),
            pl.BlockSpec((N, E), full),
            pl.BlockSpec((N, E), full),
        ],
        out_specs=blk,
        out_shape=jax.ShapeDtypeStruct((B, N, C), jnp.float32),
    )(Q, K, V, eijT, W1, b1c, W2, b2c, P0, P1, G0, G1)


def kernel(Q, K, V, eij, W1, b1, W2, b2, src, dst):
    B, N, C = Q.shape
    E = src.shape[0]
    H = W1.shape[0]

    order = jnp.argsort(dst.astype(jnp.int32))
    psrc = src.astype(jnp.int32)[order]
    p0i = psrc[0::2]
    p1i = psrc[1::2]
    e0i = order[0::2].astype(jnp.int32)
    e1i = order[1::2].astype(jnp.int32)
    idx = jnp.stack([p0i, p1i, e0i, e1i]).astype(jnp.int32)

    eijT = eij.T
    mlp = (
        jnp.zeros((8, 16), jnp.float32)
        .at[0].set(W1[:, 0])
        .at[1].set(W1[:, 1])
        .at[2].set(b1)
        .at[3].set(W2[0])
        .at[4, 0].set(b2[0])
    )

    bsc = _SC_BATCHES
    if bsc >= B:
        return _sc_call(B, Q, K, V, eijT, mlp, idx)

    ar = jnp.arange(N, dtype=jnp.int32)
    are = jnp.arange(E, dtype=jnp.int32)
    P0 = (p0i[:, None] == ar[None, :]).astype(jnp.float32)
    P1 = (p1i[:, None] == ar[None, :]).astype(jnp.float32)
    G0 = (e0i[:, None] == are[None, :]).astype(jnp.float32)
    G1 = (e1i[:, None] == are[None, :]).astype(jnp.float32)
    b1c = b1.reshape(H, 1)
    b2c = b2.reshape(1, 1)

    out_tc = _tc_call(bsc, Q, K, V, eijT, W1, b1c, W2, b2c, P0, P1, G0, G1)
    out_sc = _sc_call(bsc, Q, K, V, eijT, mlp, idx)
    return lax.dynamic_update_slice(out_tc, out_sc, (0, 0, 0))


# speedup vs baseline: 1.2135x; 1.0616x over previous
import functools
import jax
import jax.numpy as jnp
from jax import lax
from jax.experimental import pallas as pl
from jax.experimental.pallas import tpu as pltpu
from jax.experimental.pallas import tpu_sc as plsc

_NC = 2
_NS = 16
_NW = _NC * _NS
_L = 16


def _sc_body(nb, q_hbm, k_hbm, v_hbm, eijt_hbm, mlp_hbm, idx_hbm, out_hbm,
             eij_v, mlp_v, idx_v,
             qb0, qb1, qb2, kb0, kb1, vb0, vb1, sca_v,
             sq0, sq1, sq2, sk0, sk1, sv0, sv1, so0, so1, so2):
    B, N, C = q_hbm.shape
    E = eijt_hbm.shape[1]
    nch = C // _L
    bpw = nb // _NW
    wid = lax.axis_index("s") * _NC + lax.axis_index("c")
    base = wid * bpw

    qbufs = [qb0, qb1, qb2]
    kbufs = [kb0, kb1]
    vbufs = [vb0, vb1]
    sq = [sq0, sq1, sq2]
    sk = [sk0, sk1]
    sv = [sv0, sv1]
    so = [so0, so1, so2]

    pltpu.sync_copy(eijt_hbm, eij_v)
    pltpu.sync_copy(mlp_hbm, mlp_v)
    pltpu.sync_copy(idx_hbm, idx_v)

    row0 = jnp.zeros((_L,), jnp.int32)
    row1 = jnp.full((_L,), 1, jnp.int32)
    row2 = jnp.full((_L,), 2, jnp.int32)

    w1s = mlp_v[0, pl.ds(0, _L)]
    w1d = mlp_v[1, pl.ds(0, _L)]
    b1r = mlp_v[2, pl.ds(0, _L)]
    w2r = mlp_v[3, pl.ds(0, _L)]
    b2r = mlp_v[4, pl.ds(0, _L)]
    for t in range(E // _L):
        s_vec = eij_v[0, pl.ds(t * _L, _L)]
        d_vec = eij_v[1, pl.ds(t * _L, _L)]
        acc = jnp.zeros((_L,), jnp.float32) + b2r[0]
        for j in range(16):
            h = jnp.maximum(s_vec * w1s[j] + d_vec * w1d[j] + b1r[j], 0.0)
            acc = acc + h * w2r[j]
        sca_v[0, pl.ds(t * _L, _L)] = acc

    for t in range(N // _L):
        sl = pl.ds(t * _L, _L)
        sca_v[1, sl] = plsc.load_gather(sca_v, [row0, idx_v[2, sl]])
        sca_v[2, sl] = plsc.load_gather(sca_v, [row0, idx_v[3, sl]])

    def node_compute(n, qb, kb, vb):
        nsplat = jnp.full((_L,), n, jnp.int32)
        p0 = plsc.load_gather(idx_v, [row0, nsplat])[0]
        p1 = plsc.load_gather(idx_v, [row1, nsplat])[0]
        a0a = jnp.zeros((_L,), jnp.float32)
        a0b = jnp.zeros((_L,), jnp.float32)
        a1a = jnp.zeros((_L,), jnp.float32)
        a1b = jnp.zeros((_L,), jnp.float32)
        for ct in range(0, nch, 2):
            qc = qb[n, pl.ds(ct * _L, _L)]
            a0a = a0a + qc * kb[p0, pl.ds(ct * _L, _L)]
            a1a = a1a + qc * kb[p1, pl.ds(ct * _L, _L)]
            qd = qb[n, pl.ds((ct + 1) * _L, _L)]
            a0b = a0b + qd * kb[p0, pl.ds((ct + 1) * _L, _L)]
            a1b = a1b + qd * kb[p1, pl.ds((ct + 1) * _L, _L)]
        s0 = jnp.sum(a0a + a0b, axis=0)
        s1 = jnp.sum(a1a + a1b, axis=0)
        l0 = plsc.load_gather(sca_v, [row1, nsplat]) + s0
        l1 = plsc.load_gather(sca_v, [row2, nsplat]) + s1
        m = jnp.maximum(l0, l1)
        e0 = jnp.exp(l0 - m)
        e1 = jnp.exp(l1 - m)
        inv = 1.0 / (e0 + e1)
        a0 = e0 * inv
        a1 = e1 * inv
        for ct in range(nch):
            sl = pl.ds(ct * _L, _L)
            qb[n, sl] = vb[p0, sl] * a0 + vb[p1, sl] * a1

    def issue_in(bi):
        b = base + bi
        return [
            pltpu.async_copy(q_hbm.at[b], qbufs[bi % 3], sq[bi % 3]),
            pltpu.async_copy(k_hbm.at[b], kbufs[bi % 2], sk[bi % 2]),
            pltpu.async_copy(v_hbm.at[b], vbufs[bi % 2], sv[bi % 2]),
        ]

    inh = {0: issue_in(0)}
    if bpw > 1:
        inh[1] = issue_in(1)
    outh = {}
    out_waited = set()
    for bi in range(bpw):
        for h in inh.pop(bi):
            h.wait()
        qb = qbufs[bi % 3]
        kb = kbufs[bi % 2]
        vb = vbufs[bi % 2]

        @plsc.parallel_loop(0, N, 1, unroll=1)
        def _nloop(n, qb=qb, kb=kb, vb=vb):
            node_compute(n, qb, kb, vb)
        outh[bi] = pltpu.async_copy(qb, out_hbm.at[base + bi], so[bi % 3])
        if bi + 2 < bpw:
            if bi - 1 >= 0 and (bi - 1) not in out_waited:
                outh[bi - 1].wait()
                out_waited.add(bi - 1)
            inh[bi + 2] = issue_in(bi + 2)
    for bi in range(bpw):
        if bi not in out_waited:
            outh[bi].wait()


def _sc_call(nb, Q, K, V, eijT, mlp, idx):
    B, N, C = Q.shape
    E = eijT.shape[1]
    mesh = plsc.VectorSubcoreMesh(
        core_axis_name="c", subcore_axis_name="s", num_cores=_NC, num_subcores=_NS
    )
    buf = pltpu.VMEM((N, C), jnp.float32)
    return pl.kernel(
        functools.partial(_sc_body, nb),
        out_type=jax.ShapeDtypeStruct((nb, N, C), jnp.float32),
        mesh=mesh,
        compiler_params=pltpu.CompilerParams(needs_layout_passes=False),
        scratch_types=[
            pltpu.VMEM((2, E), jnp.float32),
            pltpu.VMEM((8, 16), jnp.float32),
            pltpu.VMEM((4, N), jnp.int32),
            buf, buf, buf, buf, buf, buf, buf,
            pltpu.VMEM((8, E), jnp.float32),
        ] + [pltpu.SemaphoreType.DMA] * 10,
    )(Q, K, V, eijT, mlp, idx)


_SC_BATCHES = 64


def _tc_body(qref, kref, vref, eijt, w1, b1c, w2, b2c, p0, p1, g0, g1, oref):
    h = jnp.maximum(
        jnp.dot(w1[...], eijt[...], preferred_element_type=jnp.float32) + b1c[...],
        0.0,
    )
    bias = jnp.dot(w2[...], h, preferred_element_type=jnp.float32) + b2c[...]
    bias0 = jnp.sum(g0[...] * bias, axis=1)
    bias1 = jnp.sum(g1[...] * bias, axis=1)

    q = qref[...]
    k = kref[...]
    v = vref[...]
    dn = (((2,), (2,)), ((0,), (0,)))
    s = jax.lax.dot_general(
        q, k, dn,
        precision=jax.lax.Precision.HIGHEST,
        preferred_element_type=jnp.float32,
    )
    l0 = jnp.sum(s * p0[...][None], axis=2) + bias0[None]
    l1 = jnp.sum(s * p1[...][None], axis=2) + bias1[None]
    m = jnp.maximum(l0, l1)
    e0 = jnp.exp(l0 - m)
    e1 = jnp.exp(l1 - m)
    inv = 1.0 / (e0 + e1)
    wmix = (e0 * inv)[:, :, None] * p0[...][None] + (e1 * inv)[:, :, None] * p1[...][None]
    dn2 = (((2,), (1,)), ((0,), (0,)))
    oref[...] = jax.lax.dot_general(wmix, v, dn2, preferred_element_type=jnp.float32)


def _tc_call(skip, Q, K, V, eijT, W1, b1c, W2, b2c, P0, P1, G0, G1):
    B, N, C = Q.shape
    E = eijT.shape[1]
    H = W1.shape[0]
    bs = 32
    off = skip // bs
    grid = ((B - skip) // bs,)
    full = lambda i: (0, 0)
    blk = pl.BlockSpec((bs, N, C), lambda i: (i + off, 0, 0))
    return pl.pallas_call(
        _tc_body,
        grid=grid,
        in_specs=[
            blk,
            blk,
            blk,
            pl.BlockSpec((2, E), full),
            pl.BlockSpec((H, 2), full),
            pl.BlockSpec((H, 1), full),
            pl.BlockSpec((1, H), full),
            pl.BlockSpec((1, 1), full),
            pl.BlockSpec((N, N), full),
            pl.BlockSpec((N, N), full),
            pl.BlockSpec((N, E), full),
            pl.BlockSpec((N, E), full),
        ],
        out_specs=blk,
        out_shape=jax.ShapeDtypeStruct((B, N, C), jnp.float32),
    )(Q, K, V, eijT, W1, b1c, W2, b2c, P0, P1, G0, G1)


def kernel(Q, K, V, eij, W1, b1, W2, b2, src, dst):
    B, N, C = Q.shape
    E = src.shape[0]
    H = W1.shape[0]

    order = jnp.argsort(dst.astype(jnp.int32))
    psrc = src.astype(jnp.int32)[order]
    p0i = psrc[0::2]
    p1i = psrc[1::2]
    e0i = order[0::2].astype(jnp.int32)
    e1i = order[1::2].astype(jnp.int32)
    idx = jnp.stack([p0i, p1i, e0i, e1i]).astype(jnp.int32)

    eijT = eij.T
    mlp = (
        jnp.zeros((8, 16), jnp.float32)
        .at[0].set(W1[:, 0])
        .at[1].set(W1[:, 1])
        .at[2].set(b1)
        .at[3].set(W2[0])
        .at[4, 0].set(b2[0])
    )

    bsc = _SC_BATCHES
    if bsc >= B:
        return _sc_call(B, Q, K, V, eijT, mlp, idx)

    ar = jnp.arange(N, dtype=jnp.int32)
    are = jnp.arange(E, dtype=jnp.int32)
    P0 = (p0i[:, None] == ar[None, :]).astype(jnp.float32)
    P1 = (p1i[:, None] == ar[None, :]).astype(jnp.float32)
    G0 = (e0i[:, None] == are[None, :]).astype(jnp.float32)
    G1 = (e1i[:, None] == are[None, :]).astype(jnp.float32)
    b1c = b1.reshape(H, 1)
    b2c = b2.reshape(1, 1)

    out_tc = _tc_call(bsc, Q, K, V, eijT, W1, b1c, W2, b2c, P0, P1, G0, G1)
    out_sc = _sc_call(bsc, Q, K, V, eijT, mlp, idx)
    return lax.dynamic_update_slice(out_tc, out_sc, (0, 0, 0))
